# baseline (speedup 1.0000x reference)
import functools

import jax
import jax.numpy as jnp
from jax.experimental import pallas as pl

_N = 10000
_D = 128
_K = 4
_POOL = 0.5
_MM_BLK = 400


def _with_self_loops(ei, num_nodes):
    loops = jnp.arange(num_nodes, dtype=ei.dtype)
    return jnp.concatenate([ei, jnp.stack([loops, loops])], axis=1)


def _mm_kernel(x_ref, w_ref, a2_ref, h0, h1, h2, h3, sd_ref):
    xb = x_ref[...]
    sd = jnp.zeros((_MM_BLK, _D), jnp.float32)
    for k, href in enumerate((h0, h1, h2, h3)):
        hk = jnp.dot(xb, w_ref[k], preferred_element_type=jnp.float32)
        href[...] = hk
        sd = sd + jnp.dot(hk, a2_ref[k], preferred_element_type=jnp.float32)
    sd_ref[...] = sd


@jax.jit
def _dense_stage(x, W, A2):
    grid = (_N // _MM_BLK,)
    outs = pl.pallas_call(
        _mm_kernel,
        out_shape=[jax.ShapeDtypeStruct((_N, _D), jnp.float32)] * 5,
        grid=grid,
        in_specs=[
            pl.BlockSpec((_MM_BLK, _D), lambda i: (i, 0)),
            pl.BlockSpec((_K, _D, _D), lambda i: (0, 0, 0)),
            pl.BlockSpec((_K, _D, _D), lambda i: (0, 0, 0)),
        ],
        out_specs=[pl.BlockSpec((_MM_BLK, _D), lambda i: (i, 0))] * 5,
    )(x, W, A2)
    return outs[:4], outs[4]


def _ew_call(fn, n_out, *arrs):
    e = arrs[0].shape[0]
    rows = -(-e // (256 * _D)) * 256
    ep = rows * _D
    grid = (rows // 256,)
    padded = [jnp.pad(a, (0, ep - e), constant_values=1.0).reshape(rows, _D)
              for a in arrs]
    outs = pl.pallas_call(
        fn,
        out_shape=[jax.ShapeDtypeStruct((rows, _D), jnp.float32)] * n_out,
        grid=grid,
        in_specs=[pl.BlockSpec((256, _D), lambda i: (i, 0))] * len(arrs),
        out_specs=[pl.BlockSpec((256, _D), lambda i: (i, 0))] * n_out,
    )(*padded)
    if n_out == 1:
        return outs[0].reshape(-1)[:e]
    return [o.reshape(-1)[:e] for o in outs]


def _logit_kernel(*refs):
    ins, outs = refs[:8], refs[8:]
    for k in range(_K):
        z = ins[k][...] + ins[4 + k][...]
        outs[k][...] = jnp.where(z >= 0, z, jnp.float32(0.2) * z)


def _exp_kernel(*refs):
    ins, outs = refs[:8], refs[8:]
    for k in range(_K):
        outs[k][...] = jnp.exp(ins[k][...] - ins[4 + k][...])


def _attn_kernel(*refs):
    ins, outs = refs[:8], refs[8:]
    att = []
    for k in range(_K):
        a = ins[k][...] / (ins[4 + k][...] + jnp.float32(1e-16))
        att.append(a)
        outs[k][...] = a
    outs[4][...] = (((att[0] + att[1]) + att[2]) + att[3]) / jnp.float32(4.0)


def _meag(x, ei, W, As, Ad):
    src, dst = ei[0], ei[1]
    A2 = jnp.zeros((_K, _D, _D), jnp.float32)
    for k in range(_K):
        A2 = A2.at[k, :, 2 * k].set(As[k]).at[k, :, 2 * k + 1].set(Ad[k])
    hs, sd = _dense_stage(x, W, A2)

    cols = jax.lax.optimization_barrier(tuple(sd[:, c] for c in range(2 * _K)))
    sa = [cols[2 * k][src] for k in range(_K)]
    da = [cols[2 * k + 1][dst] for k in range(_K)]
    e = _ew_call(_logit_kernel, _K, *sa, *da)

    emaxg = []
    for k in range(_K):
        emax = jax.ops.segment_max(e[k], dst, num_segments=_N)
        emax = jnp.where(jnp.isfinite(emax), emax, 0.0)
        emaxg.append(emax[dst])
    ex = _ew_call(_exp_kernel, _K, *e, *emaxg)

    dg = []
    for k in range(_K):
        denom = jax.ops.segment_sum(ex[k], dst, num_segments=_N)
        dg.append(denom[dst])
    res = _ew_call(_attn_kernel, _K + 1, *ex, *dg)
    attn, attn_mean = res[:_K], res[_K]

    outs = []
    for k in range(_K):
        outs.append(jax.ops.segment_sum(attn[k][:, None] * hs[k][src], dst,
                                        num_segments=_N))
    out = (((outs[0] + outs[1]) + outs[2]) + outs[3]) / 4.0
    return out, attn_mean


def kernel(x, edge_index, W1, W2, W3, As1, Ad1, As2, Ad2, As3, Ad3):
    params = [(W1, As1, Ad1), (W2, As2, Ad2), (W3, As3, Ad3)]
    edge_list = []
    ei = _with_self_loops(edge_index, x.shape[0])
    for i in range(3):
        edge_list.append(ei)
        x, attn = _meag(x, ei, *params[i])
        x = jax.nn.leaky_relu(x, 0.01)
        x = x / jnp.maximum(jnp.linalg.norm(x, axis=0, keepdims=True), 1e-12)
        kk = max(int(attn.shape[0] * _POOL), 1)
        _, idx = jax.lax.top_k(attn, kk)
        ei = _with_self_loops(ei[:, idx], x.shape[0])
    return (x, ei) + tuple(edge_list)


# speedup vs baseline: 3.0581x; 3.0581x over previous
import functools

import jax
import jax.numpy as jnp
from jax import lax
from jax.experimental import pallas as pl
from jax.experimental.pallas import tpu as pltpu
from jax.experimental.pallas import tpu_sc as plsc

_N = 10000
_D = 128
_K = 4
_POOL = 0.5
_MM_BLK = 400


_SC_INFO = plsc.get_sparse_core_info()
_NW = _SC_INFO.num_cores * _SC_INFO.num_subcores
_NP = 10016
_CB = 1024


@functools.partial(jax.jit, static_argnames=("ep",))
def _gather4(tab4, idx, ep):
    chunk = ep // _NW
    mesh = plsc.VectorSubcoreMesh(core_axis_name="c", subcore_axis_name="s")

    @functools.partial(
        pl.kernel, mesh=mesh,
        out_type=[jax.ShapeDtypeStruct((ep,), jnp.float32)] * 4,
        scratch_types=[pltpu.VMEM((_CB,), jnp.int32)]
        + [pltpu.VMEM((_CB,), jnp.float32)] * 4
        + [pltpu.SemaphoreType.DMA],
    )
    def k(t0h, t1h, t2h, t3h, idx_hbm, o0, o1, o2, o3,
          idx_v, b0, b1, b2, b3, sem):
        wid = lax.axis_index("s") * _SC_INFO.num_cores + lax.axis_index("c")
        base = wid * chunk
        tabs = (t0h, t1h, t2h, t3h)
        outs = (o0, o1, o2, o3)
        bufs = (b0, b1, b2, b3)

        def outer(ci, carry):
            cb = base + ci * _CB
            pltpu.sync_copy(idx_hbm.at[pl.ds(cb, _CB)], idx_v)
            handles = [pltpu.async_copy(tabs[t].at[idx_v], bufs[t], sem)
                       for t in range(4)]
            for hd in handles:
                hd.wait()
            for t in range(4):
                pltpu.sync_copy(bufs[t], outs[t].at[pl.ds(cb, _CB)])
            return carry

        lax.fori_loop(0, chunk // _CB, outer, 0)

    return k(tab4[0], tab4[1], tab4[2], tab4[3], idx)


def _sc_gather(node_arrays, idx, e):
    ep = -(-e // (_NW * _CB)) * (_NW * _CB)
    tab = tuple(jnp.pad(a, (0, _NP - _N)) for a in node_arrays)
    idxp = jnp.concatenate(
        [idx, (jnp.arange(ep - e, dtype=jnp.int32) * 61) % _N])
    out = _gather4(tab, idxp, ep)
    return [o[:e] for o in out]


def _with_self_loops(ei, num_nodes):
    loops = jnp.arange(num_nodes, dtype=ei.dtype)
    return jnp.concatenate([ei, jnp.stack([loops, loops])], axis=1)


def _mm_kernel(x_ref, w_ref, a2_ref, h0, h1, h2, h3, sd_ref):
    xb = x_ref[...]
    sd = jnp.zeros((_MM_BLK, _D), jnp.float32)
    for k, href in enumerate((h0, h1, h2, h3)):
        hk = jnp.dot(xb, w_ref[k], preferred_element_type=jnp.float32)
        href[...] = hk
        sd = sd + jnp.dot(hk, a2_ref[k], preferred_element_type=jnp.float32)
    sd_ref[...] = sd


@jax.jit
def _dense_stage(x, W, A2):
    grid = (_N // _MM_BLK,)
    outs = pl.pallas_call(
        _mm_kernel,
        out_shape=[jax.ShapeDtypeStruct((_N, _D), jnp.float32)] * 5,
        grid=grid,
        in_specs=[
            pl.BlockSpec((_MM_BLK, _D), lambda i: (i, 0)),
            pl.BlockSpec((_K, _D, _D), lambda i: (0, 0, 0)),
            pl.BlockSpec((_K, _D, _D), lambda i: (0, 0, 0)),
        ],
        out_specs=[pl.BlockSpec((_MM_BLK, _D), lambda i: (i, 0))] * 5,
    )(x, W, A2)
    return outs[:4], outs[4]


def _ew_call(fn, n_out, *arrs):
    e = arrs[0].shape[0]
    rows = -(-e // (256 * _D)) * 256
    ep = rows * _D
    grid = (rows // 256,)
    padded = [jnp.pad(a, (0, ep - e), constant_values=1.0).reshape(rows, _D)
              for a in arrs]
    outs = pl.pallas_call(
        fn,
        out_shape=[jax.ShapeDtypeStruct((rows, _D), jnp.float32)] * n_out,
        grid=grid,
        in_specs=[pl.BlockSpec((256, _D), lambda i: (i, 0))] * len(arrs),
        out_specs=[pl.BlockSpec((256, _D), lambda i: (i, 0))] * n_out,
    )(*padded)
    if n_out == 1:
        return outs[0].reshape(-1)[:e]
    return [o.reshape(-1)[:e] for o in outs]


def _logit_kernel(*refs):
    ins, outs = refs[:8], refs[8:]
    for k in range(_K):
        z = ins[k][...] + ins[4 + k][...]
        outs[k][...] = jnp.where(z >= 0, z, jnp.float32(0.2) * z)


def _exp_kernel(*refs):
    ins, outs = refs[:8], refs[8:]
    for k in range(_K):
        outs[k][...] = jnp.exp(ins[k][...] - ins[4 + k][...])


def _attn_kernel(*refs):
    ins, outs = refs[:8], refs[8:]
    att = []
    for k in range(_K):
        a = ins[k][...] / (ins[4 + k][...] + jnp.float32(1e-16))
        att.append(a)
        outs[k][...] = a
    outs[4][...] = (((att[0] + att[1]) + att[2]) + att[3]) / jnp.float32(4.0)


def _meag(x, ei, W, As, Ad):
    src, dst = ei[0], ei[1]
    A2 = jnp.zeros((_K, _D, _D), jnp.float32)
    for k in range(_K):
        A2 = A2.at[k, :, 2 * k].set(As[k]).at[k, :, 2 * k + 1].set(Ad[k])
    hs, sd = _dense_stage(x, W, A2)

    ne = int(src.shape[0])
    cols = jax.lax.optimization_barrier(tuple(sd[:, c] for c in range(2 * _K)))
    sa = _sc_gather([cols[2 * k] for k in range(_K)], src, ne)
    da = _sc_gather([cols[2 * k + 1] for k in range(_K)], dst, ne)
    e = _ew_call(_logit_kernel, _K, *sa, *da)

    emaxs = []
    for k in range(_K):
        emax = jax.ops.segment_max(e[k], dst, num_segments=_N)
        emaxs.append(jnp.where(jnp.isfinite(emax), emax, 0.0))
    emaxg = _sc_gather(emaxs, dst, ne)
    ex = _ew_call(_exp_kernel, _K, *e, *emaxg)

    dens = [jax.ops.segment_sum(ex[k], dst, num_segments=_N) for k in range(_K)]
    dg = _sc_gather(dens, dst, ne)
    res = _ew_call(_attn_kernel, _K + 1, *ex, *dg)
    attn, attn_mean = res[:_K], res[_K]

    outs = []
    for k in range(_K):
        outs.append(jax.ops.segment_sum(attn[k][:, None] * hs[k][src], dst,
                                        num_segments=_N))
    out = (((outs[0] + outs[1]) + outs[2]) + outs[3]) / 4.0
    return out, attn_mean


def kernel(x, edge_index, W1, W2, W3, As1, Ad1, As2, Ad2, As3, Ad3):
    params = [(W1, As1, Ad1), (W2, As2, Ad2), (W3, As3, Ad3)]
    edge_list = []
    ei = _with_self_loops(edge_index, x.shape[0])
    for i in range(3):
        edge_list.append(ei)
        x, attn = _meag(x, ei, *params[i])
        x = jax.nn.leaky_relu(x, 0.01)
        x = x / jnp.maximum(jnp.linalg.norm(x, axis=0, keepdims=True), 1e-12)
        kk = max(int(attn.shape[0] * _POOL), 1)
        _, idx = jax.lax.top_k(attn, kk)
        ei = _with_self_loops(ei[:, idx], x.shape[0])
    return (x, ei) + tuple(edge_list)
